# baseline (speedup 1.0000x reference)
import functools
import math

import jax
import jax.numpy as jnp
from jax.experimental import pallas as pl


def _tile_kernel(ids_ref, mask_ref, embp_ref, w1f_ref, b1f_ref, w2bd_ref,
                 b2_ref, pooled_ref, oa_ref, *, V):
    ids = ids_ref[...]
    idsb = ids.astype(jnp.bfloat16)
    maskb = mask_ref[...].astype(jnp.bfloat16)
    TB = ids.shape[0]
    Vp, D = embp_ref.shape

    lio = jax.lax.broadcasted_iota(jnp.int32, (TB, Vp), 1)
    counts = jnp.zeros((TB, Vp), jnp.float32)
    for v in range(V):
        cnt_v = jnp.sum(jnp.where(idsb == v, maskb, 0), axis=1)
        counts = counts + cnt_v.astype(jnp.float32)[:, None] * (
            lio == v).astype(jnp.float32)
    denom = jnp.clip(jnp.sum(maskb, axis=1).astype(jnp.float32), 1e-9, None)

    pooled = jnp.dot(counts, embp_ref[...],
                     preferred_element_type=jnp.float32,
                     precision=jax.lax.Precision.HIGHEST) / denom[:, None]
    pooled_ref[...] = pooled

    h = jnp.dot(pooled, w1f_ref[...], preferred_element_type=jnp.float32)
    h = jnp.maximum(h + b1f_ref[0, :][None, :], 0.0)
    out_all = jnp.dot(h, w2bd_ref[...], preferred_element_type=jnp.float32)
    oa_ref[...] = out_all + b2_ref[0, :][None, :]


def _accept_kernel(cat_ref, catt_ref, out_ref, *, capacity, chunk):
    nb = catt_ref.shape[1]
    v0r = catt_ref[0:1, :]
    v1r = catt_ref[1:2, :]
    e0r = catt_ref[2:3, :]
    e1r = catt_ref[3:4, :]
    bi = jax.lax.broadcasted_iota(jnp.int32, (1, nb), 1)
    ones_row = jnp.ones((1, chunk), jnp.float32)

    def body(c, carry):
        cnt0, cnt1 = carry
        s = c * chunk
        v0c = cat_ref[pl.ds(s, chunk), 0:1]
        v1c = cat_ref[pl.ds(s, chunk), 1:2]
        e0c = cat_ref[pl.ds(s, chunk), 2:3]
        e1c = cat_ref[pl.ds(s, chunk), 3:4]
        bj = jax.lax.broadcasted_iota(jnp.int32, (chunk, 1), 0) + s
        lt = bj < bi
        le = bj <= bi
        a00 = (e0c == e0r) & ((v0c > v0r) | ((v0c == v0r) & lt))
        a10 = (e1c == e0r) & ((v1c > v0r) | ((v1c == v0r) & lt))
        a01 = (e0c == e1r) & ((v0c > v1r) | ((v0c == v1r) & le))
        a11 = (e1c == e1r) & ((v1c > v1r) | ((v1c == v1r) & lt))
        ind0 = jnp.where(a00, 1.0, 0.0) + jnp.where(a10, 1.0, 0.0)
        ind1 = jnp.where(a01, 1.0, 0.0) + jnp.where(a11, 1.0, 0.0)
        cnt0 = cnt0 + jnp.dot(ones_row, ind0, preferred_element_type=jnp.float32)
        cnt1 = cnt1 + jnp.dot(ones_row, ind1, preferred_element_type=jnp.float32)
        return cnt0, cnt1

    z = jnp.zeros((1, nb), jnp.float32)
    cnt0, cnt1 = jax.lax.fori_loop(0, nb // chunk, body, (z, z))
    acc0 = (cnt0 < capacity).astype(jnp.float32)
    acc1 = (cnt1 < capacity).astype(jnp.float32)
    aw0 = catt_ref[4:5, :] * acc0
    aw1 = catt_ref[5:6, :] * acc1
    den = jnp.clip(aw0 + aw1, 1e-9, None)
    out_ref[...] = (aw0 * catt_ref[6:7, :] + aw1 * catt_ref[7:8, :]) / den


def kernel(input_ids, attention_mask, emb, W_r, b_r, W1, b1, W2, b2):
    B, L = input_ids.shape
    V, D = emb.shape
    E = W_r.shape[1]
    H = W1.shape[2]
    K = 2
    capacity = max(1, int(math.ceil(1.25 * B * K / E)))

    Vp = 64
    embp = jnp.zeros((Vp, D), jnp.float32).at[:V].set(emb)
    w1f = jnp.transpose(W1, (1, 0, 2)).reshape(D, E * H)
    b1f = b1.reshape(1, E * H)
    w2bd = jnp.zeros((E * H, E), jnp.float32).at[
        jnp.arange(E * H), jnp.repeat(jnp.arange(E), H)].set(W2.reshape(-1))

    TB = 128
    grid = (B // TB,)
    full = lambda shape: pl.BlockSpec(shape, lambda i: tuple(0 for _ in shape))
    pooled, out_all = pl.pallas_call(
        functools.partial(_tile_kernel, V=V),
        grid=grid,
        in_specs=[
            pl.BlockSpec((TB, L), lambda i: (i, 0)),
            pl.BlockSpec((TB, L), lambda i: (i, 0)),
            full((Vp, D)),
            full((D, E * H)),
            full((1, E * H)),
            full((E * H, E)),
            full((1, E)),
        ],
        out_specs=[pl.BlockSpec((TB, D), lambda i: (i, 0)),
                   pl.BlockSpec((TB, E), lambda i: (i, 0))],
        out_shape=[jax.ShapeDtypeStruct((B, D), jnp.float32),
                   jax.ShapeDtypeStruct((B, E), jnp.float32)],
    )(input_ids, attention_mask, embp, w1f, b1f, w2bd, b2.reshape(1, E))

    router_logits = pooled @ W_r + b_r
    router_probs = jax.nn.softmax(router_logits, axis=-1)
    topk_vals, topk_idx = jax.lax.top_k(router_probs, K)
    topk_weights = topk_vals / jnp.clip(
        jnp.sum(topk_vals, axis=-1, keepdims=True), 1e-9, None)
    ts = jnp.take_along_axis(out_all, topk_idx, axis=1)
    cat = jnp.concatenate(
        [topk_vals, topk_idx.astype(jnp.float32), topk_weights, ts], axis=1)

    scores = pl.pallas_call(
        functools.partial(_accept_kernel, capacity=capacity, chunk=256),
        in_specs=[pl.BlockSpec((B, 8), lambda: (0, 0)),
                  pl.BlockSpec((8, B), lambda: (0, 0))],
        out_specs=pl.BlockSpec((1, B), lambda: (0, 0)),
        out_shape=jax.ShapeDtypeStruct((1, B), jnp.float32),
    )(cat, cat.T)
    return scores.reshape(B)


# speedup vs baseline: 1.6703x; 1.6703x over previous
import functools
import math

import jax
import jax.numpy as jnp
from jax.experimental import pallas as pl


def _tile_kernel(ids_ref, mask_ref, embp_ref, w1f_ref, b1f_ref, w2bd_ref,
                 b2_ref, pooled_ref, oa_ref, *, V):
    ids = ids_ref[...]
    idsb = ids.astype(jnp.bfloat16)
    maskb = mask_ref[...].astype(jnp.bfloat16)
    TB = ids.shape[0]
    Vp, D = embp_ref.shape

    lio = jax.lax.broadcasted_iota(jnp.int32, (TB, Vp), 1)
    counts = jnp.zeros((TB, Vp), jnp.float32)
    for v in range(V):
        cnt_v = jnp.sum(jnp.where(idsb == v, maskb, 0), axis=1)
        counts = counts + cnt_v.astype(jnp.float32)[:, None] * (
            lio == v).astype(jnp.float32)
    denom = jnp.clip(jnp.sum(maskb, axis=1).astype(jnp.float32), 1e-9, None)

    pooled = jnp.dot(counts, embp_ref[...],
                     preferred_element_type=jnp.float32,
                     precision=jax.lax.Precision.HIGHEST) / denom[:, None]
    pooled_ref[...] = pooled

    h = jnp.dot(pooled, w1f_ref[...], preferred_element_type=jnp.float32)
    h = jnp.maximum(h + b1f_ref[0, :][None, :], 0.0)
    out_all = jnp.dot(h, w2bd_ref[...], preferred_element_type=jnp.float32)
    oa_ref[...] = out_all + b2_ref[0, :][None, :]


def _accept_kernel(probs_ref, probst_ref, oa_ref, out_ref, *, capacity, chunk):
    nb, E = probs_ref.shape
    probs = probs_ref[...]
    probst = probst_ref[...]
    out_all = oa_ref[...]

    lio = jax.lax.broadcasted_iota(jnp.int32, (nb, E), 1)
    m1c = jnp.max(probs, axis=1, keepdims=True)
    i1c = jnp.min(jnp.where(probs == m1c, lio, E), axis=1, keepdims=True)
    probs_b = jnp.where(lio == i1c, -1.0, probs)
    m2c = jnp.max(probs_b, axis=1, keepdims=True)
    i2c = jnp.min(jnp.where(probs_b == m2c, lio, E), axis=1, keepdims=True)

    sio = jax.lax.broadcasted_iota(jnp.int32, (E, nb), 0)
    m1r = jnp.max(probst, axis=0, keepdims=True)
    i1r = jnp.min(jnp.where(probst == m1r, sio, E), axis=0, keepdims=True)
    probst_b = jnp.where(sio == i1r, -1.0, probst)
    m2r = jnp.max(probst_b, axis=0, keepdims=True)
    i2r = jnp.min(jnp.where(probst_b == m2r, sio, E), axis=0, keepdims=True)

    bi = jax.lax.broadcasted_iota(jnp.int32, (nb, 1), 0)
    ones_col = jnp.ones((chunk, 1), jnp.float32)
    cnt0 = jnp.zeros((nb, 1), jnp.float32)
    cnt1 = jnp.zeros((nb, 1), jnp.float32)
    for c in range(nb // chunk):
        s = c * chunk
        v0j = m1r[:, s:s + chunk]
        v1j = m2r[:, s:s + chunk]
        e0j = i1r[:, s:s + chunk]
        e1j = i2r[:, s:s + chunk]
        bj = jax.lax.broadcasted_iota(jnp.int32, (1, chunk), 1) + s
        lt = bj < bi
        le = bj <= bi
        a00 = (e0j == i1c) & ((v0j > m1c) | ((v0j == m1c) & lt))
        a10 = (e1j == i1c) & ((v1j > m1c) | ((v1j == m1c) & lt))
        a01 = (e0j == i2c) & ((v0j > m2c) | ((v0j == m2c) & le))
        a11 = (e1j == i2c) & ((v1j > m2c) | ((v1j == m2c) & lt))
        ind0 = jnp.where(a00, 1.0, 0.0) + jnp.where(a10, 1.0, 0.0)
        ind1 = jnp.where(a01, 1.0, 0.0) + jnp.where(a11, 1.0, 0.0)
        cnt0 = cnt0 + jnp.dot(ind0, ones_col, preferred_element_type=jnp.float32)
        cnt1 = cnt1 + jnp.dot(ind1, ones_col, preferred_element_type=jnp.float32)

    acc0 = (cnt0 < capacity).astype(jnp.float32)
    acc1 = (cnt1 < capacity).astype(jnp.float32)
    wsum = jnp.clip(m1c + m2c, 1e-9, None)
    aw0 = (m1c / wsum) * acc0
    aw1 = (m2c / wsum) * acc1
    ts0 = jnp.sum(out_all * (lio == i1c).astype(jnp.float32), axis=1,
                  keepdims=True)
    ts1 = jnp.sum(out_all * (lio == i2c).astype(jnp.float32), axis=1,
                  keepdims=True)
    den = jnp.clip(aw0 + aw1, 1e-9, None)
    out_ref[...] = (aw0 * ts0 + aw1 * ts1) / den


def kernel(input_ids, attention_mask, emb, W_r, b_r, W1, b1, W2, b2):
    B, L = input_ids.shape
    V, D = emb.shape
    E = W_r.shape[1]
    H = W1.shape[2]
    K = 2
    capacity = max(1, int(math.ceil(1.25 * B * K / E)))

    Vp = 64
    embp = jnp.zeros((Vp, D), jnp.float32).at[:V].set(emb)
    w1f = jnp.transpose(W1, (1, 0, 2)).reshape(D, E * H)
    b1f = b1.reshape(1, E * H)
    w2bd = jnp.zeros((E * H, E), jnp.float32).at[
        jnp.arange(E * H), jnp.repeat(jnp.arange(E), H)].set(W2.reshape(-1))

    TB = 128
    grid = (B // TB,)
    full = lambda shape: pl.BlockSpec(shape, lambda i: tuple(0 for _ in shape))
    pooled, out_all = pl.pallas_call(
        functools.partial(_tile_kernel, V=V),
        grid=grid,
        in_specs=[
            pl.BlockSpec((TB, L), lambda i: (i, 0)),
            pl.BlockSpec((TB, L), lambda i: (i, 0)),
            full((Vp, D)),
            full((D, E * H)),
            full((1, E * H)),
            full((E * H, E)),
            full((1, E)),
        ],
        out_specs=[pl.BlockSpec((TB, D), lambda i: (i, 0)),
                   pl.BlockSpec((TB, E), lambda i: (i, 0))],
        out_shape=[jax.ShapeDtypeStruct((B, D), jnp.float32),
                   jax.ShapeDtypeStruct((B, E), jnp.float32)],
    )(input_ids, attention_mask, embp, w1f, b1f, w2bd, b2.reshape(1, E))

    router_logits = pooled @ W_r + b_r
    router_probs = jax.nn.softmax(router_logits, axis=-1)

    scores = pl.pallas_call(
        functools.partial(_accept_kernel, capacity=capacity, chunk=256),
        in_specs=[pl.BlockSpec((B, E), lambda: (0, 0)),
                  pl.BlockSpec((E, B), lambda: (0, 0)),
                  pl.BlockSpec((B, E), lambda: (0, 0))],
        out_specs=pl.BlockSpec((B, 1), lambda: (0, 0)),
        out_shape=jax.ShapeDtypeStruct((B, 1), jnp.float32),
    )(router_probs, router_probs.T, out_all)
    return scores.reshape(B)
